# baseline (speedup 1.0000x reference)
import functools

import jax
import jax.numpy as jnp
from jax import lax
from jax.experimental import pallas as pl
from jax.experimental.pallas import tpu as pltpu
from jax.experimental.pallas import tpu_sc as plsc

EMB_SIZE = 64
SCALE = 8.0
VOCAB = 1000000
CBLK = 128
W = 256
N_SLAB = (VOCAB - EMB_SIZE) // W
IPITCH = 261
GPITCH = 133


def _iota16():
    return lax.iota(jnp.int32, 16)


@functools.lru_cache(maxsize=None)
def _make_transpose_kernel():
    info = plsc.get_sparse_core_info()
    nc, ns = info.num_cores, info.num_subcores
    nw = nc * ns
    nbuf = 2
    n_rounds = -(-N_SLAB // nw)
    n_rounds += (-n_rounds) % nbuf
    n_outer = n_rounds // nbuf
    mesh = plsc.VectorSubcoreMesh(core_axis_name="c", subcore_axis_name="s")

    @functools.partial(
        pl.kernel,
        mesh=mesh,
        out_type=jax.ShapeDtypeStruct((VOCAB // 2, 2 * EMB_SIZE), jnp.float32),
        compiler_params=pltpu.CompilerParams(
            use_tc_tiling_on_sc=True, needs_layout_passes=False
        ),
        scratch_types=(
            [pltpu.VMEM((EMB_SIZE, IPITCH), jnp.float32) for _ in range(nbuf)]
            + [pltpu.VMEM((W // 2, 2 * EMB_SIZE), jnp.float32) for _ in range(nbuf)]
            + [pltpu.SemaphoreType.DMA for _ in range(2 * nbuf)]
        ),
    )
    def sc_transpose(tab_hbm, tail_hbm, packed_hbm, *scratch):
        in_buf = scratch[:nbuf]
        out_buf = scratch[nbuf : 2 * nbuf]
        isem = scratch[2 * nbuf : 3 * nbuf]
        osem = scratch[3 * nbuf : 4 * nbuf]
        wid = lax.axis_index("s") * nc + lax.axis_index("c")

        @pl.when(wid == 0)
        def _():
            pltpu.sync_copy(tail_hbm, out_buf[0].at[pl.ds(0, 32)])
            pltpu.sync_copy(
                out_buf[0].at[pl.ds(0, 32)],
                packed_hbm.at[pl.ds(VOCAB // 2 - 32, 32)],
            )

        def slab(k):
            return jnp.minimum(wid + k * nw, N_SLAB - 1)

        def loads(k, b):
            c0 = slab(k) * W
            return [
                pltpu.make_async_copy(
                    tab_hbm.at[g, :, pl.ds(c0, W)],
                    in_buf[b].at[pl.ds(8 * g, 8), pl.ds(0, W)],
                    isem[b],
                )
                for g in range(8)
            ]

        def store(k, b):
            return pltpu.make_async_copy(
                out_buf[b],
                packed_hbm.at[pl.ds(slab(k) * (W // 2), W // 2)],
                osem[b],
            )

        rowv = [_iota16() + 16 * g for g in range(4)]
        for b in range(nbuf):
            for cp in loads(b, b):
                cp.start()

        @pl.loop(0, n_outer)
        def outer(r):
            for b in range(nbuf):
                k = r * nbuf + b
                for cp in loads(k, b):
                    cp.wait()

                @pl.when(r > 0)
                def _():
                    store(k - nbuf, b).wait()

                @plsc.parallel_loop(0, W // 2)
                def transpose_row(u):
                    for h in range(2):
                        colv = jnp.full((16,), 0, jnp.int32) + (2 * u + h)
                        for g in range(4):
                            val = plsc.load_gather(in_buf[b], [rowv[g], colv])
                            out_buf[b][u, pl.ds(64 * h + 16 * g, 16)] = val

                @pl.when(r < n_outer - 1)
                def _():
                    for cp in loads(k + nbuf, b):
                        cp.start()

                store(k, b).start()

        for b in range(nbuf):
            store((n_outer - 1) * nbuf + b, b).wait()

    return sc_transpose


@functools.lru_cache(maxsize=None)
def _make_gather_kernel(bsz: int, seq: int):
    info = plsc.get_sparse_core_info()
    nc, ns = info.num_cores, info.num_subcores
    nw = nc * ns
    nbuf = 2
    assert bsz % (nw * CBLK) == 0
    n_outer = -(-seq // nbuf)
    mesh = plsc.VectorSubcoreMesh(core_axis_name="c", subcore_axis_name="s")

    @functools.partial(
        pl.kernel,
        mesh=mesh,
        out_type=jax.ShapeDtypeStruct((seq, EMB_SIZE, bsz), jnp.float32),
        compiler_params=pltpu.CompilerParams(
            use_tc_tiling_on_sc=True, needs_layout_passes=False
        ),
        scratch_types=(
            [
                pltpu.VMEM((seq, CBLK), jnp.int32),
                pltpu.VMEM((seq, CBLK), jnp.int32),
            ]
            + [pltpu.VMEM((CBLK, GPITCH), jnp.float32) for _ in range(nbuf)]
            + [pltpu.VMEM((EMB_SIZE, CBLK), jnp.float32) for _ in range(nbuf)]
            + [pltpu.SemaphoreType.DMA for _ in range(2 * nbuf)]
        ),
    )
    def sc_gather(packed_hbm, tok_hbm, out_hbm, idx_v, half_v, *scratch):
        gbuf = scratch[:nbuf]
        obuf = scratch[nbuf : 2 * nbuf]
        gsem = scratch[2 * nbuf : 3 * nbuf]
        ssem = scratch[3 * nbuf : 4 * nbuf]
        wid = lax.axis_index("s") * nc + lax.axis_index("c")
        col0 = wid * CBLK

        pltpu.sync_copy(tok_hbm.at[:, pl.ds(col0, CBLK)], idx_v)

        @pl.loop(0, seq)
        def halve(s):
            for m in range(8):
                sl = pl.ds(16 * m, 16)
                half_v[s, sl] = lax.shift_right_logical(idx_v[s, sl], 1)

        def pos(s):
            return jnp.minimum(s, seq - 1)

        def gather(s, b):
            return pltpu.make_async_copy(
                packed_hbm.at[half_v.at[pos(s)]],
                gbuf[b].at[:, pl.ds(0, 2 * EMB_SIZE)],
                gsem[b],
            )

        def store(s, b):
            return pltpu.make_async_copy(
                obuf[b], out_hbm.at[pos(s), :, pl.ds(col0, CBLK)], ssem[b]
            )

        rowv = [_iota16() + 16 * m for m in range(8)]
        for b in range(nbuf):
            gather(b, b).start()

        @pl.loop(0, n_outer)
        def outer(r):
            for b in range(nbuf):
                s = r * nbuf + b
                gather(s, b).wait()

                @pl.when(r > 0)
                def _():
                    store(s - nbuf, b).wait()

                parv = [
                    lax.shift_left(
                        lax.bitwise_and(idx_v[pos(s), pl.ds(16 * m, 16)], 1), 6
                    )
                    for m in range(8)
                ]

                @plsc.parallel_loop(0, EMB_SIZE)
                def reorder(e):
                    for m in range(8):
                        val = plsc.load_gather(gbuf[b], [rowv[m], parv[m] + e])
                        obuf[b][e, pl.ds(16 * m, 16)] = val * SCALE

                @pl.when(r < n_outer - 1)
                def _():
                    gather(s + nbuf, b).start()

                store(s, b).start()

        for b in range(nbuf):
            store((n_outer - 1) * nbuf + b, b).wait()

    return sc_gather


@jax.jit
def kernel(tokens, emb_table):
    bsz, seq = tokens.shape
    tab3 = emb_table.T.reshape(8, 8, VOCAB)
    tail = emb_table[VOCAB - 64 :].reshape(32, 2 * EMB_SIZE)
    packed = _make_transpose_kernel()(tab3, tail)
    tok_t = tokens.astype(jnp.int32).T
    out_phys = _make_gather_kernel(bsz, seq)(packed, tok_t)
    return jnp.transpose(out_phys, (2, 0, 1))
